# baseline (speedup 1.0000x reference)
import functools

import jax
import jax.numpy as jnp
from jax import lax
from jax.experimental import pallas as pl
from jax.experimental.pallas import tpu as pltpu
from jax.experimental.pallas import tpu_sc as plsc

D = 1024
E = 8
K = 2
N = 8192
C = 2048
H = 4096
LANES = 128
PAD_ROW = E * C
NW = 32
TPW = N // NW
CHUNK = 64
QPW = TPW // CHUNK

_HI_BITS = 0x3F800001


def _router_body(x_ref, wg_ref, scores_ref, inds_ref):
    x = x_ref[...]
    wg = wg_ref[...]
    logits = jnp.dot(x, wg, preferred_element_type=jnp.float32)
    lane = lax.broadcasted_iota(jnp.int32, logits.shape, 1)
    valid = lane < E
    lg = jnp.where(valid, logits, -jnp.inf)
    m = jnp.max(lg, axis=1, keepdims=True)
    ex = jnp.exp(lg - m)
    gates = ex / jnp.sum(ex, axis=1, keepdims=True)
    g = jnp.where(valid, gates, -1.0)
    m1 = jnp.max(g, axis=1, keepdims=True)
    e1 = jnp.min(jnp.where(g == m1, lane, LANES), axis=1, keepdims=True)
    g2 = jnp.where(lane == e1, -1.0, g)
    m2 = jnp.max(g2, axis=1, keepdims=True)
    e2 = jnp.min(jnp.where(g2 == m2, lane, LANES), axis=1, keepdims=True)
    top2 = (lane == e1) | (lane == e2)
    scores_ref[...] = jnp.where(top2, gates, 0.0)
    inds_ref[...] = jnp.where(lane == 0, e1, jnp.where(lane == 1, e2, 0))


def _router(flat, wg_pad):
    bm = 512
    return pl.pallas_call(
        _router_body,
        grid=(N // bm,),
        in_specs=[
            pl.BlockSpec((bm, D), lambda i: (i, 0)),
            pl.BlockSpec((D, LANES), lambda i: (0, 0)),
        ],
        out_specs=[
            pl.BlockSpec((bm, LANES), lambda i: (i, 0)),
            pl.BlockSpec((bm, LANES), lambda i: (i, 0)),
        ],
        out_shape=[
            jax.ShapeDtypeStruct((N, LANES), jnp.float32),
            jax.ShapeDtypeStruct((N, LANES), jnp.int32),
        ],
    )(flat, wg_pad)


def _excl_cumsum(src_ref, dst_ref, ltri):
    nblk = N // LANES

    def body(b, carry):
        blk = src_ref[pl.ds(b * LANES, LANES), :]
        dst_ref[pl.ds(b * LANES, LANES), :] = (
            jnp.dot(ltri, blk, preferred_element_type=jnp.float32,
                    precision=lax.Precision.HIGHEST) + carry)
        return carry + jnp.sum(blk, axis=0, keepdims=True)

    lax.fori_loop(0, nblk, body, jnp.zeros((1, LANES), jnp.float32))


def _select_body(scores_ref, inds_ref, sel_i_ref, sel_f_ref, a_ref, b_ref):
    s = scores_ref[...]
    sb = lax.bitcast_convert_type(s, jnp.int32)
    cap = jnp.int32(C)

    def bs(i, carry):
        lo, hi = carry
        mid = lax.shift_right_logical(lo + hi + 1, 1)
        cnt = jnp.sum((sb >= mid).astype(jnp.int32), axis=0, keepdims=True)
        ok = cnt >= cap
        return jnp.where(ok, mid, lo), jnp.where(ok, hi, mid - 1)

    lo0 = jnp.zeros((1, LANES), jnp.int32)
    hi0 = jnp.full((1, LANES), _HI_BITS, jnp.int32)
    t, _ = lax.fori_loop(0, 31, bs, (lo0, hi0))

    gt = sb > t
    eq = sb == t
    cnt_gt = jnp.sum(gt.astype(jnp.int32), axis=0, keepdims=True)
    need = (cap - cnt_gt).astype(jnp.float32)

    a_ref[...] = eq.astype(jnp.float32)
    ltri = (lax.broadcasted_iota(jnp.int32, (LANES, LANES), 0)
            > lax.broadcasted_iota(jnp.int32, (LANES, LANES), 1)
            ).astype(jnp.float32)
    _excl_cumsum(a_ref, b_ref, ltri)
    keep = gt | (eq & (b_ref[...] < need))
    a_ref[...] = keep.astype(jnp.float32)
    _excl_cumsum(a_ref, b_ref, ltri)
    pos = b_ref[...]

    lane = lax.broadcasted_iota(jnp.int32, s.shape, 1)
    keep_f = keep.astype(jnp.float32)

    def pick(ej):
        onehot = (lane == ej).astype(jnp.float32)
        kept = jnp.sum(onehot * keep_f, axis=1, keepdims=True) > 0.0
        p = jnp.sum(onehot * pos, axis=1, keepdims=True).astype(jnp.int32)
        gw = jnp.sum(onehot * s, axis=1, keepdims=True)
        slot = ej * C + p
        cpos_s = jnp.where(kept, slot, PAD_ROW)
        cpos_c = jnp.where(kept, slot, 0)
        gwk = jnp.where(kept, gw, 0.0)
        return cpos_s, cpos_c, gwk

    e1 = inds_ref[:, 0:1]
    e2 = inds_ref[:, 1:2]
    s0, c0, w0 = pick(e1)
    s1, c1, w1 = pick(e2)
    sel_i_ref[...] = jnp.where(
        lane == 0, s0, jnp.where(lane == 1, s1,
        jnp.where(lane == 2, c0, jnp.where(lane == 3, c1, 0))))
    sel_f_ref[...] = jnp.where(lane == 0, w0, jnp.where(lane == 1, w1, 0.0))


def _select(scores, inds):
    return pl.pallas_call(
        _select_body,
        out_shape=[
            jax.ShapeDtypeStruct((N, LANES), jnp.int32),
            jax.ShapeDtypeStruct((N, LANES), jnp.float32),
        ],
        scratch_shapes=[
            pltpu.VMEM((N, LANES), jnp.float32),
            pltpu.VMEM((N, LANES), jnp.float32),
        ],
    )(scores, inds)


PPW = 2 * N // NW
GQPW = PPW // CHUNK


@functools.cache
def _sc_kernels():
    mesh = plsc.VectorSubcoreMesh(core_axis_name="c", subcore_axis_name="s")

    @functools.partial(
        pl.kernel,
        mesh=mesh,
        out_type=jax.ShapeDtypeStruct(((E + 1) * C, D), jnp.float32),
        scratch_types=[
            pltpu.VMEM((2 * QPW, CHUNK), jnp.int32),
            pltpu.VMEM((CHUNK, D), jnp.float32),
            pltpu.SemaphoreType.DMA,
        ],
    )
    def dispatch(flat_hbm, idx_hbm, xg_hbm, idx_v, rows_v, sem):
        wid = lax.axis_index("s") * 2 + lax.axis_index("c")
        pltpu.sync_copy(idx_hbm.at[wid], idx_v)
        for q in range(QPW):
            base = wid * TPW + q * CHUNK
            pltpu.sync_copy(flat_hbm.at[pl.ds(base, CHUNK)], rows_v)
            for j in range(2):
                pltpu.async_copy(
                    rows_v, xg_hbm.at[idx_v.at[j * QPW + q]], sem).wait()

    @functools.partial(
        pl.kernel,
        mesh=mesh,
        out_type=jax.ShapeDtypeStruct((2 * N, D), jnp.float32),
        scratch_types=[
            pltpu.VMEM((GQPW, CHUNK), jnp.int32),
            pltpu.VMEM((CHUNK, D), jnp.float32),
            pltpu.SemaphoreType.DMA,
        ],
    )
    def gather(eo_hbm, idx_hbm, gath_hbm, idx_v, rows_v, sem):
        wid = lax.axis_index("s") * 2 + lax.axis_index("c")
        pltpu.sync_copy(idx_hbm.at[wid], idx_v)
        for q in range(GQPW):
            pltpu.async_copy(eo_hbm.at[idx_v.at[q]], rows_v, sem).wait()
            pltpu.sync_copy(
                rows_v, gath_hbm.at[pl.ds(wid * PPW + q * CHUNK, CHUNK)])

    return dispatch, gather


def _dispatch(flat, disp_idx):
    return _sc_kernels()[0](flat, disp_idx)


def _gather(eo, gath_idx):
    return _sc_kernels()[1](eo, gath_idx)


_SQRT_HALF = 0.7071067811865476


def _ffn_body(x_ref, w1_ref, b1_ref, w2_ref, b2_ref, out_ref):
    nt = pl.program_id(1)
    x = x_ref[...]
    w1 = w1_ref[0]
    b1 = b1_ref[0]
    pre = jnp.dot(x, w1, preferred_element_type=jnp.float32,
                  precision=lax.Precision.HIGHEST) + b1
    h = pre * 0.5 * (1.0 + lax.erf(pre * _SQRT_HALF))
    contrib = jnp.dot(h, w2_ref[0], preferred_element_type=jnp.float32,
                      precision=lax.Precision.HIGHEST)

    @pl.when(nt == 0)
    def _():
        out_ref[...] = b2_ref[0] + contrib

    @pl.when(nt != 0)
    def _():
        out_ref[...] += contrib


def _ffn(xg, w1, b1, w2, b2):
    ht = 512
    return pl.pallas_call(
        _ffn_body,
        grid=(E, H // ht),
        in_specs=[
            pl.BlockSpec((C, D), lambda e, nt: (e, 0)),
            pl.BlockSpec((1, D, ht), lambda e, nt: (e, 0, nt)),
            pl.BlockSpec((1, 1, ht), lambda e, nt: (e, 0, nt)),
            pl.BlockSpec((1, ht, D), lambda e, nt: (e, nt, 0)),
            pl.BlockSpec((1, 1, D), lambda e, nt: (e, 0, 0)),
        ],
        out_specs=pl.BlockSpec((C, D), lambda e, nt: (e, 0)),
        out_shape=jax.ShapeDtypeStruct((E * C, D), jnp.float32),
        compiler_params=pltpu.CompilerParams(
            dimension_semantics=("arbitrary", "arbitrary"),
            vmem_limit_bytes=100 * 1024 * 1024),
    )(xg, w1, b1, w2, b2)


def _combine_body(g_ref, w_ref, out_ref):
    gw0 = w_ref[:, 0:1]
    gw1 = w_ref[:, 1:2]
    c0 = jnp.where(gw0 != 0.0, gw0 * g_ref[0], 0.0)
    c1 = jnp.where(gw1 != 0.0, gw1 * g_ref[1], 0.0)
    out_ref[...] = c0 + c1


def _combine(gath, sel_f):
    bm = 512
    return pl.pallas_call(
        _combine_body,
        grid=(N // bm,),
        in_specs=[
            pl.BlockSpec((2, bm, D), lambda i: (0, i, 0)),
            pl.BlockSpec((bm, LANES), lambda i: (i, 0)),
        ],
        out_specs=pl.BlockSpec((bm, D), lambda i: (i, 0)),
        out_shape=jax.ShapeDtypeStruct((N, D), jnp.float32),
    )(gath, sel_f)


def kernel(hidden_states, Wg, W1, b1, W2, b2):
    Bq, Tq, d = hidden_states.shape
    flat = hidden_states.reshape(N, D)
    wg_pad = jnp.pad(Wg, ((0, 0), (0, LANES - E)))

    scores, inds = _router(flat, wg_pad)
    sel_i, sel_f = _select(scores, inds)

    cpos_s = jnp.transpose(sel_i[:, 0:2], (1, 0))
    disp_idx = (cpos_s.reshape(2, NW, QPW, CHUNK)
                .transpose(1, 0, 2, 3).reshape(NW, 2 * QPW, CHUNK))
    cpos_c = jnp.transpose(sel_i[:, 2:4], (1, 0))
    gath_idx = cpos_c.reshape(NW, GQPW, CHUNK)

    xg = _dispatch(flat, disp_idx)
    eo = _ffn(xg, W1, b1.reshape(E, 1, H), W2, b2.reshape(E, 1, D))
    gath = _gather(eo, gath_idx)
    final = _combine(gath.reshape(2, N, D), sel_f)

    aux = jnp.zeros((), jnp.float32)
    return final.reshape(Bq, Tq, d), aux


# speedup vs baseline: 3.3206x; 3.3206x over previous
import functools

import jax
import jax.numpy as jnp
from jax import lax
from jax.experimental import pallas as pl
from jax.experimental.pallas import tpu as pltpu
from jax.experimental.pallas import tpu_sc as plsc

D = 1024
E = 8
K = 2
N = 8192
C = 2048
H = 4096
LANES = 128
PAD_ROW = E * C
NW = 32
TPW = N // NW
CHUNK = 64
QPW = TPW // CHUNK

_HI_BITS = 0x3F800001


def _router_body(x_ref, wg_ref, scores_ref, inds_ref):
    x = x_ref[...]
    wg = wg_ref[...]
    logits = jnp.dot(x, wg, preferred_element_type=jnp.float32)
    lane = lax.broadcasted_iota(jnp.int32, logits.shape, 1)
    valid = lane < E
    lg = jnp.where(valid, logits, -jnp.inf)
    m = jnp.max(lg, axis=1, keepdims=True)
    ex = jnp.exp(lg - m)
    gates = ex / jnp.sum(ex, axis=1, keepdims=True)
    g = jnp.where(valid, gates, -1.0)
    m1 = jnp.max(g, axis=1, keepdims=True)
    e1 = jnp.min(jnp.where(g == m1, lane, LANES), axis=1, keepdims=True)
    g2 = jnp.where(lane == e1, -1.0, g)
    m2 = jnp.max(g2, axis=1, keepdims=True)
    e2 = jnp.min(jnp.where(g2 == m2, lane, LANES), axis=1, keepdims=True)
    top2 = (lane == e1) | (lane == e2)
    scores_ref[...] = jnp.where(top2, gates, 0.0)
    inds_ref[...] = jnp.where(lane == 0, e1, jnp.where(lane == 1, e2, 0))


def _router(flat, wg_pad):
    bm = 512
    return pl.pallas_call(
        _router_body,
        grid=(N // bm,),
        in_specs=[
            pl.BlockSpec((bm, D), lambda i: (i, 0)),
            pl.BlockSpec((D, LANES), lambda i: (0, 0)),
        ],
        out_specs=[
            pl.BlockSpec((bm, LANES), lambda i: (i, 0)),
            pl.BlockSpec((bm, LANES), lambda i: (i, 0)),
        ],
        out_shape=[
            jax.ShapeDtypeStruct((N, LANES), jnp.float32),
            jax.ShapeDtypeStruct((N, LANES), jnp.int32),
        ],
    )(flat, wg_pad)


def _excl_cumsum(src_ref, dst_ref, ltri):
    nblk = N // LANES

    def body(b, carry):
        blk = src_ref[pl.ds(b * LANES, LANES), :]
        dst_ref[pl.ds(b * LANES, LANES), :] = (
            jnp.dot(ltri, blk, preferred_element_type=jnp.float32,
                    precision=lax.Precision.HIGHEST) + carry)
        return carry + jnp.sum(blk, axis=0, keepdims=True)

    lax.fori_loop(0, nblk, body, jnp.zeros((1, LANES), jnp.float32))


def _select_body(scores_ref, inds_ref, sel_i_ref, sel_f_ref, a_ref, b_ref):
    s = scores_ref[...]
    sb = lax.bitcast_convert_type(s, jnp.int32)
    cap = jnp.int32(C)

    def bs(i, carry):
        lo, hi = carry
        mid = lax.shift_right_logical(lo + hi + 1, 1)
        cnt = jnp.sum((sb >= mid).astype(jnp.int32), axis=0, keepdims=True)
        ok = cnt >= cap
        return jnp.where(ok, mid, lo), jnp.where(ok, hi, mid - 1)

    lo0 = jnp.zeros((1, LANES), jnp.int32)
    hi0 = jnp.full((1, LANES), _HI_BITS, jnp.int32)
    t, _ = lax.fori_loop(0, 31, bs, (lo0, hi0))

    gt = sb > t
    eq = sb == t
    cnt_gt = jnp.sum(gt.astype(jnp.int32), axis=0, keepdims=True)
    need = (cap - cnt_gt).astype(jnp.float32)

    a_ref[...] = eq.astype(jnp.float32)
    ltri = (lax.broadcasted_iota(jnp.int32, (LANES, LANES), 0)
            > lax.broadcasted_iota(jnp.int32, (LANES, LANES), 1)
            ).astype(jnp.float32)
    _excl_cumsum(a_ref, b_ref, ltri)
    keep = gt | (eq & (b_ref[...] < need))
    a_ref[...] = keep.astype(jnp.float32)
    _excl_cumsum(a_ref, b_ref, ltri)
    pos = b_ref[...]

    lane = lax.broadcasted_iota(jnp.int32, s.shape, 1)
    keep_f = keep.astype(jnp.float32)

    def pick(ej):
        onehot = (lane == ej).astype(jnp.float32)
        kept = jnp.sum(onehot * keep_f, axis=1, keepdims=True) > 0.0
        p = jnp.sum(onehot * pos, axis=1, keepdims=True).astype(jnp.int32)
        gw = jnp.sum(onehot * s, axis=1, keepdims=True)
        slot = ej * C + p
        cpos_s = jnp.where(kept, slot, PAD_ROW)
        cpos_c = jnp.where(kept, slot, 0)
        gwk = jnp.where(kept, gw, 0.0)
        return cpos_s, cpos_c, gwk

    e1 = inds_ref[:, 0:1]
    e2 = inds_ref[:, 1:2]
    s0, c0, w0 = pick(e1)
    s1, c1, w1 = pick(e2)
    sel_i_ref[...] = jnp.where(
        lane == 0, s0, jnp.where(lane == 1, s1,
        jnp.where(lane == 2, c0, jnp.where(lane == 3, c1, 0))))
    sel_f_ref[...] = jnp.where(lane == 0, w0, jnp.where(lane == 1, w1, 0.0))


def _select(scores, inds):
    return pl.pallas_call(
        _select_body,
        out_shape=[
            jax.ShapeDtypeStruct((N, LANES), jnp.int32),
            jax.ShapeDtypeStruct((N, LANES), jnp.float32),
        ],
        scratch_shapes=[
            pltpu.VMEM((N, LANES), jnp.float32),
            pltpu.VMEM((N, LANES), jnp.float32),
        ],
    )(scores, inds)


PPW = 2 * N // NW
GQPW = PPW // CHUNK


@functools.cache
def _sc_kernels():
    mesh = plsc.VectorSubcoreMesh(core_axis_name="c", subcore_axis_name="s")

    @functools.partial(
        pl.kernel,
        mesh=mesh,
        out_type=jax.ShapeDtypeStruct(((E + 1) * C, D), jnp.float32),
        scratch_types=[
            pltpu.VMEM((2 * QPW, CHUNK), jnp.int32),
            pltpu.VMEM((CHUNK, D), jnp.float32),
            pltpu.SemaphoreType.DMA,
        ],
    )
    def dispatch(flat_hbm, idx_hbm, xg_hbm, idx_v, rows_v, sem):
        wid = lax.axis_index("s") * 2 + lax.axis_index("c")
        pltpu.sync_copy(idx_hbm.at[wid], idx_v)
        for q in range(QPW):
            base = wid * TPW + q * CHUNK
            pltpu.sync_copy(flat_hbm.at[pl.ds(base, CHUNK)], rows_v)
            for j in range(2):
                pltpu.async_copy(
                    rows_v, xg_hbm.at[idx_v.at[j * QPW + q]], sem).wait()

    @functools.partial(
        pl.kernel,
        mesh=mesh,
        out_type=jax.ShapeDtypeStruct((2 * N, D), jnp.float32),
        scratch_types=[
            pltpu.VMEM((GQPW, CHUNK), jnp.int32),
            pltpu.VMEM((CHUNK, D), jnp.float32),
            pltpu.SemaphoreType.DMA,
        ],
    )
    def gather(eo_hbm, idx_hbm, gath_hbm, idx_v, rows_v, sem):
        wid = lax.axis_index("s") * 2 + lax.axis_index("c")
        pltpu.sync_copy(idx_hbm.at[wid], idx_v)
        for q in range(GQPW):
            pltpu.async_copy(eo_hbm.at[idx_v.at[q]], rows_v, sem).wait()
            pltpu.sync_copy(
                rows_v, gath_hbm.at[pl.ds(wid * PPW + q * CHUNK, CHUNK)])

    return dispatch, gather


def _dispatch(flat, disp_idx):
    return _sc_kernels()[0](flat, disp_idx)


def _gather(eo, gath_idx):
    return _sc_kernels()[1](eo, gath_idx)


_SQRT_HALF = 0.7071067811865476


def _ffn_body(x_ref, w1_ref, b1_ref, w2_ref, b2_ref, out_ref):
    nt = pl.program_id(1)
    x = x_ref[...]
    w1 = w1_ref[0]
    b1 = b1_ref[0]
    pre = jnp.dot(x, w1, preferred_element_type=jnp.float32) + b1
    h = pre * 0.5 * (1.0 + lax.erf(pre * _SQRT_HALF))
    contrib = jnp.dot(h, w2_ref[0], preferred_element_type=jnp.float32)

    @pl.when(nt == 0)
    def _():
        out_ref[...] = b2_ref[0] + contrib

    @pl.when(nt != 0)
    def _():
        out_ref[...] += contrib


def _ffn(xg, w1, b1, w2, b2):
    ht = 512
    return pl.pallas_call(
        _ffn_body,
        grid=(E, H // ht),
        in_specs=[
            pl.BlockSpec((C, D), lambda e, nt: (e, 0)),
            pl.BlockSpec((1, D, ht), lambda e, nt: (e, 0, nt)),
            pl.BlockSpec((1, 1, ht), lambda e, nt: (e, 0, nt)),
            pl.BlockSpec((1, ht, D), lambda e, nt: (e, nt, 0)),
            pl.BlockSpec((1, 1, D), lambda e, nt: (e, 0, 0)),
        ],
        out_specs=pl.BlockSpec((C, D), lambda e, nt: (e, 0)),
        out_shape=jax.ShapeDtypeStruct((E * C, D), jnp.float32),
        compiler_params=pltpu.CompilerParams(
            dimension_semantics=("arbitrary", "arbitrary"),
            vmem_limit_bytes=100 * 1024 * 1024),
    )(xg, w1, b1, w2, b2)


def _combine_body(g_ref, w_ref, out_ref):
    gw0 = w_ref[:, 0:1]
    gw1 = w_ref[:, 1:2]
    c0 = jnp.where(gw0 != 0.0, gw0 * g_ref[0], 0.0)
    c1 = jnp.where(gw1 != 0.0, gw1 * g_ref[1], 0.0)
    out_ref[...] = c0 + c1


def _combine(gath, sel_f):
    bm = 512
    return pl.pallas_call(
        _combine_body,
        grid=(N // bm,),
        in_specs=[
            pl.BlockSpec((2, bm, D), lambda i: (0, i, 0)),
            pl.BlockSpec((bm, LANES), lambda i: (i, 0)),
        ],
        out_specs=pl.BlockSpec((bm, D), lambda i: (i, 0)),
        out_shape=jax.ShapeDtypeStruct((N, D), jnp.float32),
    )(gath, sel_f)


def kernel(hidden_states, Wg, W1, b1, W2, b2):
    Bq, Tq, d = hidden_states.shape
    flat = hidden_states.reshape(N, D)
    wg_pad = jnp.pad(Wg, ((0, 0), (0, LANES - E)))

    scores, inds = _router(flat, wg_pad)
    sel_i, sel_f = _select(scores, inds)

    cpos_s = jnp.transpose(sel_i[:, 0:2], (1, 0))
    disp_idx = (cpos_s.reshape(2, NW, QPW, CHUNK)
                .transpose(1, 0, 2, 3).reshape(NW, 2 * QPW, CHUNK))
    cpos_c = jnp.transpose(sel_i[:, 2:4], (1, 0))
    gath_idx = cpos_c.reshape(NW, GQPW, CHUNK)

    xg = _dispatch(flat, disp_idx)
    eo = _ffn(xg, W1, b1.reshape(E, 1, H), W2, b2.reshape(E, 1, D))
    gath = _gather(eo, gath_idx)
    final = _combine(gath.reshape(2, N, D), sel_f)

    aux = jnp.zeros((), jnp.float32)
    return final.reshape(Bq, Tq, d), aux


# speedup vs baseline: 3.4853x; 1.0496x over previous
import functools

import jax
import jax.numpy as jnp
from jax import lax
from jax.experimental import pallas as pl
from jax.experimental.pallas import tpu as pltpu
from jax.experimental.pallas import tpu_sc as plsc

D = 1024
E = 8
K = 2
N = 8192
C = 2048
H = 4096
LANES = 128
PAD_ROW = E * C
NW = 32
TPW = N // NW
CHUNK = 64
QPW = TPW // CHUNK

_HI_BITS = 0x3F800001


def _router_body(x_ref, wg_ref, scores_ref, inds_ref):
    x = x_ref[...]
    wg = wg_ref[...]
    logits = jnp.dot(x, wg, preferred_element_type=jnp.float32)
    lane = lax.broadcasted_iota(jnp.int32, logits.shape, 1)
    valid = lane < E
    lg = jnp.where(valid, logits, -jnp.inf)
    m = jnp.max(lg, axis=1, keepdims=True)
    ex = jnp.exp(lg - m)
    gates = ex / jnp.sum(ex, axis=1, keepdims=True)
    g = jnp.where(valid, gates, -1.0)
    m1 = jnp.max(g, axis=1, keepdims=True)
    e1 = jnp.min(jnp.where(g == m1, lane, LANES), axis=1, keepdims=True)
    g2 = jnp.where(lane == e1, -1.0, g)
    m2 = jnp.max(g2, axis=1, keepdims=True)
    e2 = jnp.min(jnp.where(g2 == m2, lane, LANES), axis=1, keepdims=True)
    top2 = (lane == e1) | (lane == e2)
    scores_ref[...] = jnp.where(top2, gates, 0.0)
    inds_ref[...] = jnp.where(lane == 0, e1, jnp.where(lane == 1, e2, 0))


def _router(flat, wg_pad):
    bm = 512
    return pl.pallas_call(
        _router_body,
        grid=(N // bm,),
        in_specs=[
            pl.BlockSpec((bm, D), lambda i: (i, 0)),
            pl.BlockSpec((D, LANES), lambda i: (0, 0)),
        ],
        out_specs=[
            pl.BlockSpec((bm, LANES), lambda i: (i, 0)),
            pl.BlockSpec((bm, LANES), lambda i: (i, 0)),
        ],
        out_shape=[
            jax.ShapeDtypeStruct((N, LANES), jnp.float32),
            jax.ShapeDtypeStruct((N, LANES), jnp.int32),
        ],
    )(flat, wg_pad)


def _excl_cumsum(src_ref, dst_ref, ltri):
    nblk = N // LANES

    def body(b, carry):
        blk = src_ref[pl.ds(b * LANES, LANES), :]
        dst_ref[pl.ds(b * LANES, LANES), :] = (
            jnp.dot(ltri, blk, preferred_element_type=jnp.float32,
                    precision=lax.Precision.HIGHEST) + carry)
        return carry + jnp.sum(blk, axis=0, keepdims=True)

    lax.fori_loop(0, nblk, body, jnp.zeros((1, LANES), jnp.float32))


def _select_body(scores_ref, inds_ref, sel_i_ref, sel_f_ref, a_ref, b_ref):
    s = scores_ref[...]
    sb = lax.bitcast_convert_type(s, jnp.int32)
    cap = jnp.int32(C)

    def bs(i, carry):
        lo, hi = carry
        mid = lax.shift_right_logical(lo + hi + 1, 1)
        cnt = jnp.sum((sb >= mid).astype(jnp.int32), axis=0, keepdims=True)
        ok = cnt >= cap
        return jnp.where(ok, mid, lo), jnp.where(ok, hi, mid - 1)

    lo0 = jnp.zeros((1, LANES), jnp.int32)
    hi0 = jnp.full((1, LANES), _HI_BITS, jnp.int32)
    t, _ = lax.fori_loop(0, 31, bs, (lo0, hi0))

    gt = sb > t
    eq = sb == t
    cnt_gt = jnp.sum(gt.astype(jnp.int32), axis=0, keepdims=True)
    need = (cap - cnt_gt).astype(jnp.float32)

    a_ref[...] = eq.astype(jnp.float32)
    ltri = (lax.broadcasted_iota(jnp.int32, (LANES, LANES), 0)
            > lax.broadcasted_iota(jnp.int32, (LANES, LANES), 1)
            ).astype(jnp.float32)
    _excl_cumsum(a_ref, b_ref, ltri)
    keep = gt | (eq & (b_ref[...] < need))
    a_ref[...] = keep.astype(jnp.float32)
    _excl_cumsum(a_ref, b_ref, ltri)
    pos = b_ref[...]

    lane = lax.broadcasted_iota(jnp.int32, s.shape, 1)
    keep_f = keep.astype(jnp.float32)

    def pick(ej):
        onehot = (lane == ej).astype(jnp.float32)
        kept = jnp.sum(onehot * keep_f, axis=1, keepdims=True) > 0.0
        p = jnp.sum(onehot * pos, axis=1, keepdims=True).astype(jnp.int32)
        gw = jnp.sum(onehot * s, axis=1, keepdims=True)
        slot = ej * C + p
        cpos_s = jnp.where(kept, slot, PAD_ROW)
        cpos_c = jnp.where(kept, slot, 0)
        gwk = jnp.where(kept, gw, 0.0)
        return cpos_s, cpos_c, gwk

    e1 = inds_ref[:, 0:1]
    e2 = inds_ref[:, 1:2]
    s0, c0, w0 = pick(e1)
    s1, c1, w1 = pick(e2)
    sel_i_ref[...] = jnp.where(
        lane == 0, s0, jnp.where(lane == 1, s1,
        jnp.where(lane == 2, c0, jnp.where(lane == 3, c1, 0))))
    sel_f_ref[...] = jnp.where(lane == 0, w0, jnp.where(lane == 1, w1, 0.0))


def _select(scores, inds):
    return pl.pallas_call(
        _select_body,
        out_shape=[
            jax.ShapeDtypeStruct((N, LANES), jnp.int32),
            jax.ShapeDtypeStruct((N, LANES), jnp.float32),
        ],
        scratch_shapes=[
            pltpu.VMEM((N, LANES), jnp.float32),
            pltpu.VMEM((N, LANES), jnp.float32),
        ],
    )(scores, inds)


PPW = 2 * N // NW
GQPW = PPW // CHUNK


@functools.cache
def _sc_kernels():
    mesh = plsc.VectorSubcoreMesh(core_axis_name="c", subcore_axis_name="s")

    @functools.partial(
        pl.kernel,
        mesh=mesh,
        out_type=jax.ShapeDtypeStruct(((E + 1) * C, D), jnp.float32),
        scratch_types=[
            pltpu.VMEM((2 * QPW, CHUNK), jnp.int32),
            pltpu.VMEM((CHUNK, D), jnp.float32),
            pltpu.SemaphoreType.DMA,
        ],
    )
    def dispatch(flat_hbm, idx_hbm, xg_hbm, idx_v, rows_v, sem):
        wid = lax.axis_index("s") * 2 + lax.axis_index("c")
        pltpu.sync_copy(idx_hbm.at[wid], idx_v)
        for q in range(QPW):
            base = wid * TPW + q * CHUNK
            pltpu.sync_copy(flat_hbm.at[pl.ds(base, CHUNK)], rows_v)
            for j in range(2):
                pltpu.async_copy(
                    rows_v, xg_hbm.at[idx_v.at[j * QPW + q]], sem).wait()

    @functools.partial(
        pl.kernel,
        mesh=mesh,
        out_type=jax.ShapeDtypeStruct((2 * N, D), jnp.float32),
        scratch_types=[
            pltpu.VMEM((GQPW, CHUNK), jnp.int32),
            pltpu.VMEM((CHUNK, D), jnp.float32),
            pltpu.SemaphoreType.DMA,
        ],
    )
    def gather(eo_hbm, idx_hbm, gath_hbm, idx_v, rows_v, sem):
        wid = lax.axis_index("s") * 2 + lax.axis_index("c")
        pltpu.sync_copy(idx_hbm.at[wid], idx_v)
        for q in range(GQPW):
            pltpu.async_copy(eo_hbm.at[idx_v.at[q]], rows_v, sem).wait()
            pltpu.sync_copy(
                rows_v, gath_hbm.at[pl.ds(wid * PPW + q * CHUNK, CHUNK)])

    return dispatch, gather


def _dispatch(flat, disp_idx):
    return _sc_kernels()[0](flat, disp_idx)


def _gather(eo, gath_idx):
    return _sc_kernels()[1](eo, gath_idx)


_SQRT_HALF = 0.7071067811865476


def _ffn_body(x_ref, w1_ref, b1_ref, w2_ref, b2_ref, out_ref):
    nt = pl.program_id(1)
    x = x_ref[...]
    w1 = w1_ref[0]
    b1 = b1_ref[0]
    pre = jnp.dot(x, w1, preferred_element_type=jnp.float32) + b1
    h = pre * 0.5 * (1.0 + lax.erf(pre * _SQRT_HALF))
    contrib = jnp.dot(h, w2_ref[0], preferred_element_type=jnp.float32)

    @pl.when(nt == 0)
    def _():
        out_ref[...] = b2_ref[0] + contrib

    @pl.when(nt != 0)
    def _():
        out_ref[...] += contrib


def _ffn(xg, w1, b1, w2, b2):
    ht = 1024
    return pl.pallas_call(
        _ffn_body,
        grid=(E, H // ht),
        in_specs=[
            pl.BlockSpec((C, D), lambda e, nt: (e, 0)),
            pl.BlockSpec((1, D, ht), lambda e, nt: (e, 0, nt)),
            pl.BlockSpec((1, 1, ht), lambda e, nt: (e, 0, nt)),
            pl.BlockSpec((1, ht, D), lambda e, nt: (e, nt, 0)),
            pl.BlockSpec((1, 1, D), lambda e, nt: (e, 0, 0)),
        ],
        out_specs=pl.BlockSpec((C, D), lambda e, nt: (e, 0)),
        out_shape=jax.ShapeDtypeStruct((E * C, D), jnp.float32),
        compiler_params=pltpu.CompilerParams(
            dimension_semantics=("arbitrary", "arbitrary"),
            vmem_limit_bytes=100 * 1024 * 1024),
    )(xg, w1, b1, w2, b2)


def _combine_body(g_ref, w_ref, out_ref):
    gw0 = w_ref[:, 0:1]
    gw1 = w_ref[:, 1:2]
    c0 = jnp.where(gw0 != 0.0, gw0 * g_ref[0], 0.0)
    c1 = jnp.where(gw1 != 0.0, gw1 * g_ref[1], 0.0)
    out_ref[...] = c0 + c1


def _combine(gath, sel_f):
    bm = 512
    return pl.pallas_call(
        _combine_body,
        grid=(N // bm,),
        in_specs=[
            pl.BlockSpec((2, bm, D), lambda i: (0, i, 0)),
            pl.BlockSpec((bm, LANES), lambda i: (i, 0)),
        ],
        out_specs=pl.BlockSpec((bm, D), lambda i: (i, 0)),
        out_shape=jax.ShapeDtypeStruct((N, D), jnp.float32),
    )(gath, sel_f)


def kernel(hidden_states, Wg, W1, b1, W2, b2):
    Bq, Tq, d = hidden_states.shape
    flat = hidden_states.reshape(N, D)
    wg_pad = jnp.pad(Wg, ((0, 0), (0, LANES - E)))

    scores, inds = _router(flat, wg_pad)
    sel_i, sel_f = _select(scores, inds)

    cpos_s = jnp.transpose(sel_i[:, 0:2], (1, 0))
    disp_idx = (cpos_s.reshape(2, NW, QPW, CHUNK)
                .transpose(1, 0, 2, 3).reshape(NW, 2 * QPW, CHUNK))
    cpos_c = jnp.transpose(sel_i[:, 2:4], (1, 0))
    gath_idx = cpos_c.reshape(NW, GQPW, CHUNK)

    xg = _dispatch(flat, disp_idx)
    eo = _ffn(xg, W1, b1.reshape(E, 1, H), W2, b2.reshape(E, 1, D))
    gath = _gather(eo, gath_idx)
    final = _combine(gath.reshape(2, N, D), sel_f)

    aux = jnp.zeros((), jnp.float32)
    return final.reshape(Bq, Tq, d), aux
